# baseline (speedup 1.0000x reference)
import jax
import jax.numpy as jnp
from jax import lax
from jax.experimental import pallas as pl
from jax.experimental.pallas import tpu as pltpu
from jax.experimental.pallas import tpu_sc as plsc

ENTITY_COUNT = 1000000
RELATION_COUNT = 1000
EMBED_DIM = 64
BATCH = 16384

NC = 2
NS = 16
L = 16
NW = NC * NS
ROWS_PER_W = BATCH // NW
CHUNK = 256
N_CHUNKS = ROWS_PER_W // CHUNK
PACKED = 2 * EMBED_DIM


def _complex_body(s_hbm, r_hbm, o_hbm, e_hbm, rel_hbm,
                  out_hbm,
                  s_v, r_v, o_v,
                  sbuf, obuf, rbuf,
                  out_v, sem):
    wid = lax.axis_index("s") * NC + lax.axis_index("c")
    base = wid * ROWS_PER_W

    pltpu.sync_copy(s_hbm.at[pl.ds(base, ROWS_PER_W)], s_v)
    pltpu.sync_copy(r_hbm.at[pl.ds(base, ROWS_PER_W)], r_v)
    pltpu.sync_copy(o_hbm.at[pl.ds(base, ROWS_PER_W)], o_v)

    iota16 = lax.iota(jnp.int32, L)

    for ci in range(N_CHUNKS):
        copies = [
            pltpu.async_copy(
                e_hbm.at[s_v.at[pl.ds(ci * CHUNK, CHUNK)]], sbuf, sem),
            pltpu.async_copy(
                e_hbm.at[o_v.at[pl.ds(ci * CHUNK, CHUNK)]], obuf, sem),
            pltpu.async_copy(
                rel_hbm.at[r_v.at[pl.ds(ci * CHUNK, CHUNK)]], rbuf, sem),
        ]
        for cp in copies:
            cp.wait()

        def group_body(g, _, ci=ci):
            rows = g * L + iota16

            def col_body(c, acc):
                re_c = jnp.zeros((L,), jnp.int32) + c
                im_c = re_c + EMBED_DIM
                sre = plsc.load_gather(sbuf, [rows, re_c])
                sim = plsc.load_gather(sbuf, [rows, im_c])
                ore = plsc.load_gather(obuf, [rows, re_c])
                oim = plsc.load_gather(obuf, [rows, im_c])
                rre = plsc.load_gather(rbuf, [rows, re_c])
                rim = plsc.load_gather(rbuf, [rows, im_c])
                return acc + ((sre * ore + sim * oim) * rre
                              + (sre * oim - sim * ore) * rim)

            acc = lax.fori_loop(0, EMBED_DIM, col_body,
                                jnp.zeros((L,), jnp.float32))
            out_v[pl.ds(ci * CHUNK + g * L, L)] = acc
            return 0

        lax.fori_loop(0, CHUNK // L, group_body, 0)

    pltpu.sync_copy(out_v, out_hbm.at[pl.ds(base, ROWS_PER_W)])


@jax.jit
def _complex_score(s, r, o, E_im, R_im, E_re, R_re):
    e_packed = jnp.concatenate([E_re, E_im], axis=1)
    rel_packed = jnp.concatenate([R_re, R_im], axis=1)
    mesh = plsc.VectorSubcoreMesh(core_axis_name="c", subcore_axis_name="s",
                                  num_cores=NC, num_subcores=NS)
    kern = pl.kernel(
        _complex_body,
        out_type=jax.ShapeDtypeStruct((BATCH,), jnp.float32),
        mesh=mesh,
        scratch_types=[
            pltpu.VMEM((ROWS_PER_W,), jnp.int32),
            pltpu.VMEM((ROWS_PER_W,), jnp.int32),
            pltpu.VMEM((ROWS_PER_W,), jnp.int32),
            pltpu.VMEM((CHUNK, PACKED), jnp.float32),
            pltpu.VMEM((CHUNK, PACKED), jnp.float32),
            pltpu.VMEM((CHUNK, PACKED), jnp.float32),
            pltpu.VMEM((ROWS_PER_W,), jnp.float32),
            pltpu.SemaphoreType.DMA,
        ],
        compiler_params=pltpu.CompilerParams(needs_layout_passes=False),
    )
    return kern(s, r, o, e_packed, rel_packed)


def kernel(s, r, o, E_im, R_im, E_re, R_re):
    s = s.astype(jnp.int32)
    r = r.astype(jnp.int32)
    o = o.astype(jnp.int32)
    return _complex_score(s, r, o, E_im, R_im, E_re, R_re)
